# baseline (speedup 1.0000x reference)
import jax
import jax.numpy as jnp
from jax.experimental import pallas as pl

NU = 25000
NI = 25000
NN = NU + NI
NE = 400000
D = 64
NNE = NN + NE


def _gcn(x, ei, W, b, n):
    src = ei[0]
    dst = ei[1]
    deg = jnp.zeros((n,), x.dtype).at[dst].add(1.0) + 1.0
    dinv = jax.lax.rsqrt(deg)
    h = x @ W
    norm = dinv[src] * dinv[dst]
    msg = jnp.take(h, src, axis=0) * norm[:, None]
    out = jnp.zeros((n, W.shape[1]), x.dtype).at[dst].add(msg)
    out = out + h * (1.0 / deg)[:, None]
    return out + b


def _dot_kernel(gu_ref, gi_ref, o_ref):
    o_ref[...] = jnp.sum(gu_ref[...] * gi_ref[...], axis=1)


def kernel(Gu, Gi, Ge, Wpn0, bpn0, Wpn1, bpn1, Wpe0, bpe0, Wpe1, bpe1, Wnn0, bnn0, Wnn1, bnn1, Wee0, bee0, Wee1, bee1, Wne0, bne0, Wne1, bne1, edge_index, node_edge_index, edge_edge_index):
    nn_emb = jnp.concatenate([Gu, Gi], axis=0)
    ee_emb = Ge
    nn_proj = jax.nn.relu(jax.nn.relu(nn_emb @ Wpn0 + bpn0) @ Wpn1 + bpn1)
    ee_proj = jax.nn.relu(jax.nn.relu(ee_emb @ Wpe0 + bpe0) @ Wpe1 + bpe1)
    ne_emb = jnp.concatenate([nn_proj, ee_proj], axis=0)
    Wnn = [(Wnn0, bnn0), (Wnn1, bnn1)]
    Wee = [(Wee0, bee0), (Wee1, bee1)]
    Wne = [(Wne0, bne0), (Wne1, bne1)]
    for l in range(2):
        nn_emb = _gcn(nn_emb, edge_index, Wnn[l][0], Wnn[l][1], NN)
        ee_emb = _gcn(ee_emb, edge_edge_index, Wee[l][0], Wee[l][1], NE)
        ne_emb = _gcn(ne_emb, node_edge_index, Wne[l][0], Wne[l][1], NNE)
        ne_node = ne_emb[:NN]
        ne_edge = ne_emb[NN:]
        nn_emb = jnp.concatenate([nn_emb, ne_node], axis=1)
        ee_emb = jnp.concatenate([ee_emb, ne_edge], axis=1)
        ne_emb = jnp.concatenate([nn_emb, ee_emb], axis=0)
    gu = nn_emb[:NU]
    gi = nn_emb[NU:]
    xui = pl.pallas_call(
        _dot_kernel,
        out_shape=jax.ShapeDtypeStruct((NU,), jnp.float32),
        grid=(1,),
        in_specs=[
            pl.BlockSpec((NU, 128), lambda i: (0, 0)),
            pl.BlockSpec((NU, 128), lambda i: (0, 0)),
        ],
        out_specs=pl.BlockSpec((NU,), lambda i: (0,)),
    )(gu, gi)
    return xui


# speedup vs baseline: 2.5130x; 2.5130x over previous
import functools

import jax
import jax.numpy as jnp
from jax import lax
from jax.experimental import pallas as pl
from jax.experimental.pallas import tpu as pltpu
from jax.experimental.pallas import tpu_sc as plsc

NU = 25000
NI = 25000
NN = NU + NI
NE = 400000
D = 64
E_NN = 800000
E_EE = 800000
E_NE = 1600000
NNE = NN + NE

L = 16
NSC = 2
NT = 16

BLOCK = 29696
BLOCK_PAD = 29824
DUMP = BLOCK
CH = 2000
CH_D = 5000
BATCH = 80
NV = CH // L

_mesh = plsc.VectorSubcoreMesh(core_axis_name="c", subcore_axis_name="s")


def _pad_rows(x, r):
    return jnp.pad(x, ((0, r - x.shape[0]),) + ((0, 0),) * (x.ndim - 1))


def _static_chunks(total, maxc):
    out, off = [], 0
    while off < total:
        sz = min(maxc, total - off)
        out.append((off, sz))
        off += sz
    return out


def _deg_body(nchunks_sc, dst_hbm, ones_hbm, zeros_hbm, out_hbm,
              acc, dstbuf, ones, zbuf):
    c = lax.axis_index("c")
    s = lax.axis_index("s")
    n_pad = acc.shape[0]
    npad16 = n_pad // NT
    off = s * npad16
    pltpu.sync_copy(ones_hbm, ones)
    pltpu.sync_copy(zeros_hbm, zbuf)
    for o, sz in _static_chunks(npad16, 1024):
        pltpu.sync_copy(zbuf.at[pl.ds(0, sz)], acc.at[pl.ds(off + o, sz)])
    plsc.subcore_barrier()

    def chunk(i, _):
        g = i * (NSC * NT) + s * NSC + c
        pltpu.sync_copy(dst_hbm.at[pl.ds(g * CH_D, CH_D)], dstbuf)
        pltpu.sync_copy(ones, acc.at[dstbuf], add=True)
        return 0

    lax.fori_loop(0, nchunks_sc, chunk, 0)
    plsc.subcore_barrier()
    for o, sz in _static_chunks(npad16, CH_D):
        pltpu.sync_copy(acc.at[pl.ds(off + o, sz)], ones.at[pl.ds(0, sz)])
        pltpu.sync_copy(ones.at[pl.ds(0, sz)],
                        out_hbm.at[pl.ds(c * n_pad + off + o, sz)])


def _deg_call(dst, n_pad):
    e = dst.shape[0]
    nchunks_sc = e // CH_D // (NSC * NT)
    assert nchunks_sc * CH_D * NSC * NT == e
    body = functools.partial(_deg_body, nchunks_sc)
    k = pl.kernel(
        body,
        out_type=jax.ShapeDtypeStruct((NSC * n_pad,), jnp.float32),
        mesh=_mesh,
        scratch_types=[
            pltpu.VMEM_SHARED((n_pad,), jnp.float32),
            pltpu.VMEM((CH_D,), jnp.int32),
            pltpu.VMEM((CH_D,), jnp.float32),
            pltpu.VMEM((1024,), jnp.float32),
        ],
    )
    parts = k(dst, jnp.ones((CH_D,), jnp.float32),
              jnp.zeros((1024,), jnp.float32))
    return parts.reshape(NSC, n_pad)


def _seg_body(K, nchunks_tile, hp_hbm, src_hbm, dst_hbm, zeros_hbm, out_hbm,
              acc, srcbuf, dstbuf, ldstbuf, sel_sd, sel_dd, rows, gsem):
    c = lax.axis_index("c")
    s = lax.axis_index("s")
    tpr = BLOCK_PAD // NT
    wpr = BLOCK // NT

    def block(j, _):
        k = c + 2 * j
        lo = k * BLOCK

        pltpu.sync_copy(zeros_hbm, rows)
        for zo, zs in _static_chunks(tpr, BATCH):
            pltpu.sync_copy(rows.at[pl.ds(0, zs)],
                            acc.at[pl.ds(s * tpr + zo, zs)])
        plsc.subcore_barrier()

        def chunk(i, _):
            g = i * NT + s
            pltpu.sync_copy(src_hbm.at[pl.ds(g * CH, CH)], srcbuf)
            pltpu.sync_copy(dst_hbm.at[pl.ds(g * CH, CH)], dstbuf)

            def vreg(v, _):
                dv = dstbuf[pl.ds(v * L, L)]
                m = (dv >= lo) & (dv < lo + BLOCK)
                ld = dv - lo
                ldstbuf[pl.ds(v * L, L)] = jnp.where(
                    m, ld, jnp.full((L,), DUMP, jnp.int32))
                return 0

            lax.fori_loop(0, NV, vreg, 0)

            def dr(b, _):
                boff = b * BATCH

                def cp(i2, _):
                    sel_sd[pl.ds(i2 * L, L)] = srcbuf[pl.ds(boff + i2 * L,
                                                            L)]
                    sel_dd[pl.ds(i2 * L, L)] = ldstbuf[pl.ds(boff + i2 * L,
                                                             L)]
                    return 0

                lax.fori_loop(0, BATCH // L, cp, 0)
                pltpu.async_copy(hp_hbm.at[sel_sd], rows, gsem).wait()
                pltpu.sync_copy(rows, acc.at[sel_dd], add=True)
                return 0

            lax.fori_loop(0, CH // BATCH, dr, 0)
            return 0

        lax.fori_loop(0, nchunks_tile, chunk, 0)
        plsc.subcore_barrier()
        base = s * wpr
        for o, sz in _static_chunks(wpr, BATCH):
            pltpu.sync_copy(acc.at[pl.ds(base + o, sz)],
                            rows.at[pl.ds(0, sz)])
            pltpu.sync_copy(rows.at[pl.ds(0, sz)],
                            out_hbm.at[pl.ds(lo + base + o, sz)])
        plsc.subcore_barrier()
        return 0

    nb = (K + 1 - c) // 2
    lax.fori_loop(0, nb, block, 0)


_SEG_STUB = False


def _seg_call(hp, src, dst, K):
    if _SEG_STUB:
        return jnp.zeros((K * BLOCK, D), jnp.float32).at[dst].add(
            jnp.take(hp, src, axis=0))
    e = src.shape[0]
    nchunks_tile = e // CH // NT
    assert nchunks_tile * CH * NT == e
    body = functools.partial(_seg_body, K, nchunks_tile)
    k = pl.kernel(
        body,
        out_type=jax.ShapeDtypeStruct((K * BLOCK, D), jnp.float32),
        mesh=_mesh,
        compiler_params=pltpu.CompilerParams(use_tc_tiling_on_sc=False),
        scratch_types=[
            pltpu.VMEM_SHARED((BLOCK_PAD, D), jnp.float32),
            pltpu.VMEM((CH,), jnp.int32),
            pltpu.VMEM((CH,), jnp.int32),
            pltpu.VMEM((CH,), jnp.int32),
            pltpu.VMEM((BATCH,), jnp.int32),
            pltpu.VMEM((BATCH,), jnp.int32),
            pltpu.VMEM((BATCH, D), jnp.float32),
            pltpu.SemaphoreType.DMA,
        ],
    )
    return k(hp, src, dst, jnp.zeros((BATCH, D), jnp.float32))


_RB = 1024


def _proj_k(x_ref, w0_ref, b0_ref, w1_ref, b1_ref, o_ref):
    h = jnp.dot(x_ref[...], w0_ref[...], preferred_element_type=jnp.float32)
    h = jnp.maximum(h + b0_ref[...][None, :], 0.0)
    o = jnp.dot(h, w1_ref[...], preferred_element_type=jnp.float32)
    o_ref[...] = jnp.maximum(o + b1_ref[...][None, :], 0.0)


def _proj(x, w0, b0, w1, b1):
    r = x.shape[0]
    return pl.pallas_call(
        _proj_k,
        out_shape=jax.ShapeDtypeStruct((r, D), jnp.float32),
        grid=(r // _RB,),
        in_specs=[
            pl.BlockSpec((_RB, D), lambda i: (i, 0)),
            pl.BlockSpec((D, D), lambda i: (0, 0)),
            pl.BlockSpec((D,), lambda i: (0,)),
            pl.BlockSpec((D, D), lambda i: (0, 0)),
            pl.BlockSpec((D,), lambda i: (0,)),
        ],
        out_specs=pl.BlockSpec((_RB, D), lambda i: (i, 0)),
    )(x, w0, b0, w1, b1)


def _dinv_k(p_ref, o_ref):
    o_ref[...] = lax.rsqrt(p_ref[0, :] + p_ref[1, :] + 1.0)


def _dinv(parts):
    n = parts.shape[1]
    return pl.pallas_call(
        _dinv_k,
        out_shape=jax.ShapeDtypeStruct((n,), jnp.float32),
        grid=(n // _RB,),
        in_specs=[pl.BlockSpec((NSC, _RB), lambda i: (0, i))],
        out_specs=pl.BlockSpec((_RB,), lambda i: (i,)),
    )(parts)


def _hp_k(x_ref, dinv_ref, w_ref, o_ref):
    xs = x_ref[...] * dinv_ref[...][:, None]
    o_ref[...] = jnp.dot(xs, w_ref[...], preferred_element_type=jnp.float32)


def _hp(x, dinv, w):
    r, din = x.shape
    return pl.pallas_call(
        _hp_k,
        out_shape=jax.ShapeDtypeStruct((r, D), jnp.float32),
        grid=(r // _RB,),
        in_specs=[
            pl.BlockSpec((_RB, din), lambda i: (i, 0)),
            pl.BlockSpec((_RB,), lambda i: (i,)),
            pl.BlockSpec((din, D), lambda i: (0, 0)),
        ],
        out_specs=pl.BlockSpec((_RB, D), lambda i: (i, 0)),
    )(x, dinv, w)


def _epi_k(acc_ref, hp_ref, dinv_ref, b_ref, o_ref):
    o_ref[...] = (acc_ref[...] + hp_ref[...]) * dinv_ref[...][:, None] \
        + b_ref[...][None, :]


def _epi(acc, hp, dinv, b):
    r = hp.shape[0]
    return pl.pallas_call(
        _epi_k,
        out_shape=jax.ShapeDtypeStruct((r, D), jnp.float32),
        grid=(r // _RB,),
        in_specs=[
            pl.BlockSpec((_RB, D), lambda i: (i, 0)),
            pl.BlockSpec((_RB, D), lambda i: (i, 0)),
            pl.BlockSpec((_RB,), lambda i: (i,)),
            pl.BlockSpec((D,), lambda i: (0,)),
        ],
        out_specs=pl.BlockSpec((_RB, D), lambda i: (i, 0)),
    )(acc, hp, dinv, b)


def _dot_k(gu_ref, gi_ref, o_ref):
    o_ref[...] = jnp.sum(gu_ref[...] * gi_ref[...], axis=1)


def _final_dot(gu, gi):
    return pl.pallas_call(
        _dot_k,
        out_shape=jax.ShapeDtypeStruct((NU,), jnp.float32),
        grid=(1,),
        in_specs=[
            pl.BlockSpec((NU, 2 * D), lambda i: (0, 0)),
            pl.BlockSpec((NU, 2 * D), lambda i: (0, 0)),
        ],
        out_specs=pl.BlockSpec((NU,), lambda i: (0,)),
    )(gu, gi)


def _ceil_to(n, m):
    return (n + m - 1) // m * m


def kernel(Gu, Gi, Ge, Wpn0, bpn0, Wpn1, bpn1, Wpe0, bpe0, Wpe1, bpe1, Wnn0, bnn0, Wnn1, bnn1, Wee0, bee0, Wee1, bee1, Wne0, bne0, Wne1, bne1, edge_index, node_edge_index, edge_edge_index):
    p_nn = _ceil_to(NN, _RB)
    p_ee = _ceil_to(NE, _RB)
    p_ne = _ceil_to(NNE, _RB)
    k_nn = -(-NN // BLOCK)
    k_ee = -(-NE // BLOCK)
    k_ne = -(-NNE // BLOCK)

    src_nn, dst_nn = edge_index[0], edge_index[1]
    src_ee, dst_ee = edge_edge_index[0], edge_edge_index[1]
    src_ne, dst_ne = node_edge_index[0], node_edge_index[1]

    dinv_nn = _dinv(_deg_call(dst_nn, p_nn))
    dinv_ee = _dinv(_deg_call(dst_ee, p_ee))
    dinv_ne = _dinv(_deg_call(dst_ne, p_ne))

    nn0 = _pad_rows(jnp.concatenate([Gu, Gi], axis=0), p_nn)
    nn_proj = _proj(nn0, Wpn0, bpn0, Wpn1, bpn1)
    ee_proj = _proj(_pad_rows(Ge, p_ee), Wpe0, bpe0, Wpe1, bpe1)
    ne0 = _pad_rows(
        jnp.concatenate([nn_proj[:NN], ee_proj[:NE]], axis=0), p_ne)

    hp_nn = _hp(nn0, dinv_nn, Wnn0)
    acc_nn = _seg_call(hp_nn, src_nn, dst_nn, k_nn)
    out_nn0 = _epi(acc_nn[:p_nn], hp_nn, dinv_nn, bnn0)

    hp_ee = _hp(_pad_rows(Ge, p_ee), dinv_ee, Wee0)
    acc_ee = _seg_call(hp_ee, src_ee, dst_ee, k_ee)
    out_ee0 = _epi(acc_ee[:p_ee], hp_ee, dinv_ee, bee0)

    hp_ne = _hp(ne0, dinv_ne, Wne0)
    acc_ne = _seg_call(hp_ne, src_ne, dst_ne, k_ne)
    out_ne0 = _epi(acc_ne[:p_ne], hp_ne, dinv_ne, bne0)

    nn1 = jnp.concatenate([out_nn0[:NN], out_ne0[:NN]], axis=1)
    ee1 = jnp.concatenate([out_ee0[:NE], out_ne0[NN:NNE]], axis=1)
    ne1 = jnp.concatenate([nn1, ee1], axis=0)

    hp_nn1 = _hp(_pad_rows(nn1, p_nn), dinv_nn, Wnn1)
    acc_nn1 = _seg_call(hp_nn1, src_nn, dst_nn, k_nn)
    out_nn1 = _epi(acc_nn1[:p_nn], hp_nn1, dinv_nn, bnn1)

    hp_ne1 = _hp(_pad_rows(ne1, p_ne), dinv_ne, Wne1)
    acc_ne1 = _seg_call(hp_ne1, src_ne, dst_ne, k_nn)
    out_ne1 = _epi(acc_ne1[:p_nn], hp_ne1[:p_nn], dinv_ne[:p_nn], bne1)

    nn2 = jnp.concatenate([out_nn1[:NN], out_ne1[:NN]], axis=1)
    return _final_dot(nn2[:NU], nn2[NU:NN])
